# baseline (speedup 1.0000x reference)
import jax
import jax.numpy as jnp
from jax.experimental import pallas as pl
from jax.experimental.pallas import tpu as pltpu


def _copy_kernel(flat_ref, rs_ref, flat_out, rs_out, sem_flat, sem_rs):
    flat_copy = pltpu.make_async_copy(flat_ref, flat_out, sem_flat)
    rs_copy = pltpu.make_async_copy(rs_ref, rs_out, sem_rs)
    flat_copy.start()
    rs_copy.start()
    flat_copy.wait()
    rs_copy.wait()


def kernel(flat, row_splits):
    return pl.pallas_call(
        _copy_kernel,
        out_shape=(
            jax.ShapeDtypeStruct(flat.shape, flat.dtype),
            jax.ShapeDtypeStruct(row_splits.shape, row_splits.dtype),
        ),
        in_specs=[
            pl.BlockSpec(memory_space=pltpu.MemorySpace.HBM),
            pl.BlockSpec(memory_space=pltpu.MemorySpace.HBM),
        ],
        out_specs=(
            pl.BlockSpec(memory_space=pltpu.MemorySpace.HBM),
            pl.BlockSpec(memory_space=pltpu.MemorySpace.HBM),
        ),
        scratch_shapes=[pltpu.SemaphoreType.DMA, pltpu.SemaphoreType.DMA],
    )(flat, row_splits)


# speedup vs baseline: 1.0017x; 1.0017x over previous
import jax
import jax.numpy as jnp
from jax.experimental import pallas as pl
from jax.experimental.pallas import tpu as pltpu


_N_CHUNKS = 16


def _copy_kernel(flat_ref, rs_ref, flat_out, rs_out, sems, sem_rs):
    rs_copy = pltpu.make_async_copy(rs_ref, rs_out, sem_rs)
    rs_copy.start()
    chunk = flat_ref.shape[0] // _N_CHUNKS
    copies = []
    for i in range(_N_CHUNKS):
        c = pltpu.make_async_copy(
            flat_ref.at[pl.ds(i * chunk, chunk), :],
            flat_out.at[pl.ds(i * chunk, chunk), :],
            sems.at[i],
        )
        c.start()
        copies.append(c)
    for c in copies:
        c.wait()
    rs_copy.wait()


def kernel(flat, row_splits):
    return pl.pallas_call(
        _copy_kernel,
        out_shape=(
            jax.ShapeDtypeStruct(flat.shape, flat.dtype),
            jax.ShapeDtypeStruct(row_splits.shape, row_splits.dtype),
        ),
        in_specs=[
            pl.BlockSpec(memory_space=pltpu.MemorySpace.HBM),
            pl.BlockSpec(memory_space=pltpu.MemorySpace.HBM),
        ],
        out_specs=(
            pl.BlockSpec(memory_space=pltpu.MemorySpace.HBM),
            pl.BlockSpec(memory_space=pltpu.MemorySpace.HBM),
        ),
        scratch_shapes=[
            pltpu.SemaphoreType.DMA((_N_CHUNKS,)),
            pltpu.SemaphoreType.DMA,
        ],
    )(flat, row_splits)


# speedup vs baseline: 46.6544x; 46.5773x over previous
import jax
import jax.numpy as jnp
from jax.experimental import pallas as pl
from jax.experimental.pallas import tpu as pltpu

_BLOCK = 2048


def _copy_kernel(flat_ref, rs_ref, flat_out, rs_out):
    flat_out[...] = flat_ref[...]

    @pl.when(pl.program_id(0) == 0)
    def _():
        for i in range(rs_ref.shape[0]):
            rs_out[i] = rs_ref[i]


def kernel(flat, row_splits):
    n_rows, n_feat = flat.shape
    grid = (n_rows // _BLOCK,)
    return pl.pallas_call(
        _copy_kernel,
        grid=grid,
        out_shape=(
            jax.ShapeDtypeStruct(flat.shape, flat.dtype),
            jax.ShapeDtypeStruct(row_splits.shape, row_splits.dtype),
        ),
        in_specs=[
            pl.BlockSpec((_BLOCK, n_feat), lambda i: (i, 0)),
            pl.BlockSpec(memory_space=pltpu.MemorySpace.SMEM),
        ],
        out_specs=(
            pl.BlockSpec((_BLOCK, n_feat), lambda i: (i, 0)),
            pl.BlockSpec(memory_space=pltpu.MemorySpace.SMEM),
        ),
    )(flat, row_splits)
